# baseline (speedup 1.0000x reference)
import jax
import jax.numpy as jnp
from jax.experimental import pallas as pl

B = 32
BEAM = 16
V = 32768
K = 16
EOS_INDEX = 2
EOS_THRESHOLD = 1.5
NEG_BIG = -1e20
NEG_SENT = -3e38
BIG_I = 2 ** 30

BP = 4
SEG = 1
RS = 256
NB = BP * BEAM
SEG_SHIFT = (RS * 128).bit_length() - 1


def _beam_step_kernel(lp_ref, alive_ref, vals_ref, beam_ref, tok_ref):
    x = lp_ref[...]
    alive = jnp.max(alive_ref[...].reshape(NB, 128), axis=1,
                    keepdims=True)[:, :, None]

    sio4 = jax.lax.broadcasted_iota(jnp.int32, (NB, SEG, RS, 128), 1)
    rio4 = jax.lax.broadcasted_iota(jnp.int32, (NB, SEG, RS, 128), 2)
    lio4 = jax.lax.broadcasted_iota(jnp.int32, (NB, SEG, RS, 128), 3)
    pm = (sio4 == 0) & (rio4 == 0) & (lio4 == EOS_INDEX)

    c0x = jnp.max(jnp.where(pm, jnp.float32(NEG_SENT), x), axis=2)
    m0 = jnp.max(c0x, axis=(1, 2), keepdims=True)[:, :, 0, None]
    lane3 = jax.lax.broadcasted_iota(jnp.int32, (NB, 1, 128), 2)
    eosrow = x[:, 0, 0:1, :]
    eos_x = jnp.max(jnp.where(lane3 == EOS_INDEX, eosrow,
                              jnp.float32(NEG_SENT)),
                    axis=2, keepdims=True)
    m = jnp.maximum(m0, eos_x)

    se = jnp.sum(jnp.exp(x - m[:, :, :, None]), axis=(1, 2, 3),
                 keepdims=True)
    logse = jnp.log(se[:, :, :, 0])
    lse = m + logse
    shift = alive - lse

    cond = ((eos_x - m) - logse) > (jnp.float32(EOS_THRESHOLD) * (-logse))
    eos_new = jnp.where(cond, eos_x, jnp.float32(NEG_BIG) - shift)
    lp_ref[:, 0, 0:1, :] = jnp.where(lane3 == EOS_INDEX,
                                     jnp.broadcast_to(eos_new, (NB, 1, 128)),
                                     eosrow)

    sio2 = jax.lax.broadcasted_iota(jnp.int32, (NB, SEG, 128), 1)
    lio2 = jax.lax.broadcasted_iota(jnp.int32, (NB, SEG, 128), 2)
    c_raw = jnp.where((sio2 == 0) & (lio2 == EOS_INDEX),
                      jnp.maximum(c0x, eos_new), c0x)

    x2 = lp_ref[...]
    bio4 = jax.lax.broadcasted_iota(jnp.int32, (NB, SEG, RS, 128), 0) \
        & (BEAM - 1)
    flat4 = bio4 * V + sio4 * (RS * 128) + rio4 * 128 + lio4
    f_tab = jnp.min(jnp.where(x2 == c_raw[:, :, None, :], flat4, BIG_I),
                    axis=2)
    c_tab = c_raw + shift

    cs = [c_tab[b * BEAM:(b + 1) * BEAM] for b in range(BP)]
    fs = [f_tab[b * BEAM:(b + 1) * BEAM] for b in range(BP)]

    oiota = jax.lax.broadcasted_iota(jnp.int32, (1, K), 1)
    i16 = jax.lax.broadcasted_iota(jnp.int32, (BEAM, 1, 1), 0)
    is4 = jax.lax.broadcasted_iota(jnp.int32, (1, SEG, 1), 1)
    rowsNB = jax.lax.broadcasted_iota(jnp.int32, (NB, 1, 1), 0)
    l1 = jax.lax.broadcasted_iota(jnp.int32, (1, 128), 1)
    r2 = jax.lax.broadcasted_iota(jnp.int32, (1, RS, 128), 1)
    l2 = jax.lax.broadcasted_iota(jnp.int32, (1, RS, 128), 2)
    accs = [[jnp.zeros((1, K), jnp.float32), jnp.zeros((1, K), jnp.int32),
             jnp.zeros((1, K), jnp.int32)] for _ in range(BP)]
    for i in range(K):
        for b in range(BP):
            g = jnp.max(cs[b])
            fi = jnp.min(jnp.where(cs[b] == g, fs[b], BIG_I))
            bi = fi >> 15
            v = fi & (V - 1)
            s = v >> SEG_SHIFT
            r = (v >> 7) & (RS - 1)
            lw = v & 127
            bq = b * BEAM + bi
            accs[b][0] = jnp.where(oiota == i, g, accs[b][0])
            accs[b][1] = jnp.where(oiota == i, bi, accs[b][1])
            accs[b][2] = jnp.where(oiota == i, v, accs[b][2])
            row = lp_ref[bq, s, pl.ds(r, 1), :]
            lp_ref[bq, s, pl.ds(r, 1), :] = jnp.where(
                l1 == lw, jnp.float32(NEG_SENT), row)
            xq = lp_ref[bq, pl.ds(s, 1)]
            crow = jnp.max(xq, axis=1)
            frow = jnp.min(jnp.where(xq == crow[:, None, :],
                                     bi * V + s * (RS * 128) + r2 * 128 + l2,
                                     BIG_I), axis=1)
            shift_b = jnp.max(jnp.where(rowsNB == bq, shift,
                                        jnp.float32(NEG_SENT)))
            sel = (i16 == bi) & (is4 == s)
            cs[b] = jnp.where(sel, (crow + shift_b)[:, None, :], cs[b])
            fs[b] = jnp.where(sel, frow[:, None, :], fs[b])

    for b in range(BP):
        vals_ref[b] = accs[b][0]
        beam_ref[b] = accs[b][1]
        tok_ref[b] = accs[b][2]


def kernel(log_probs, alive_scores, k):
    del k
    lp4 = log_probs.reshape(B * BEAM, SEG, RS, 128)
    alive3 = jnp.broadcast_to(alive_scores[..., None], (B, BEAM, 128))
    out_shapes = [
        jax.ShapeDtypeStruct((B, 1, K), jnp.float32),
        jax.ShapeDtypeStruct((B, 1, K), jnp.int32),
        jax.ShapeDtypeStruct((B, 1, K), jnp.int32),
    ]
    out_spec = pl.BlockSpec((BP, 1, K), lambda b: (b, 0, 0))
    topv, beam_idx, tok_idx = pl.pallas_call(
        _beam_step_kernel,
        grid=(B // BP,),
        in_specs=[
            pl.BlockSpec((NB, SEG, RS, 128), lambda b: (b, 0, 0, 0)),
            pl.BlockSpec((BP, BEAM, 128), lambda b: (b, 0, 0)),
        ],
        out_specs=[out_spec, out_spec, out_spec],
        out_shape=out_shapes,
    )(lp4, alive3)
    return (topv.reshape(B, K), beam_idx.reshape(B, K), tok_idx.reshape(B, K))


# speedup vs baseline: 1.0475x; 1.0475x over previous
import jax
import jax.numpy as jnp
from jax.experimental import pallas as pl

B = 32
BEAM = 16
V = 32768
K = 16
EOS_INDEX = 2
EOS_THRESHOLD = 1.5
NEG_BIG = -1e20
NEG_SENT = -3e38
BIG_I = 2 ** 30

BP = 4
SEG = 2
RS = 128
NB = BP * BEAM
SEG_SHIFT = (RS * 128).bit_length() - 1


def _beam_step_kernel(lp_ref, alive_ref, vals_ref, beam_ref, tok_ref):
    x = lp_ref[...]
    alive = jnp.max(alive_ref[...].reshape(NB, 128), axis=1,
                    keepdims=True)[:, :, None]

    sio4 = jax.lax.broadcasted_iota(jnp.int32, (NB, SEG, RS, 128), 1)
    rio4 = jax.lax.broadcasted_iota(jnp.int32, (NB, SEG, RS, 128), 2)
    lio4 = jax.lax.broadcasted_iota(jnp.int32, (NB, SEG, RS, 128), 3)
    pm = (sio4 == 0) & (rio4 == 0) & (lio4 == EOS_INDEX)

    c0x = jnp.max(jnp.where(pm, jnp.float32(NEG_SENT), x), axis=2)
    m0 = jnp.max(c0x, axis=(1, 2), keepdims=True)[:, :, 0, None]
    lane3 = jax.lax.broadcasted_iota(jnp.int32, (NB, 1, 128), 2)
    eosrow = x[:, 0, 0:1, :]
    eos_x = jnp.max(jnp.where(lane3 == EOS_INDEX, eosrow,
                              jnp.float32(NEG_SENT)),
                    axis=2, keepdims=True)
    m = jnp.maximum(m0, eos_x)

    se = jnp.sum(jnp.exp(x - m[:, :, :, None]), axis=(1, 2, 3),
                 keepdims=True)
    logse = jnp.log(se[:, :, :, 0])
    lse = m + logse
    shift = alive - lse

    cond = ((eos_x - m) - logse) > (jnp.float32(EOS_THRESHOLD) * (-logse))
    eos_new = jnp.where(cond, eos_x, jnp.float32(NEG_BIG) - shift)
    lp_ref[:, 0, 0:1, :] = jnp.where(lane3 == EOS_INDEX,
                                     jnp.broadcast_to(eos_new, (NB, 1, 128)),
                                     eosrow)

    sio2 = jax.lax.broadcasted_iota(jnp.int32, (NB, SEG, 128), 1)
    lio2 = jax.lax.broadcasted_iota(jnp.int32, (NB, SEG, 128), 2)
    c_raw = jnp.where((sio2 == 0) & (lio2 == EOS_INDEX),
                      jnp.maximum(c0x, eos_new), c0x)

    x2 = lp_ref[...]
    bio4 = jax.lax.broadcasted_iota(jnp.int32, (NB, SEG, RS, 128), 0) \
        & (BEAM - 1)
    flat4 = bio4 * V + sio4 * (RS * 128) + rio4 * 128 + lio4
    f_tab = jnp.min(jnp.where(x2 == c_raw[:, :, None, :], flat4, BIG_I),
                    axis=2)
    c_tab = c_raw + shift

    cs = [c_tab[b * BEAM:(b + 1) * BEAM] for b in range(BP)]
    fs = [f_tab[b * BEAM:(b + 1) * BEAM] for b in range(BP)]

    oiota = jax.lax.broadcasted_iota(jnp.int32, (1, K), 1)
    i16 = jax.lax.broadcasted_iota(jnp.int32, (BEAM, 1, 1), 0)
    is4 = jax.lax.broadcasted_iota(jnp.int32, (1, SEG, 1), 1)
    rowsNB = jax.lax.broadcasted_iota(jnp.int32, (NB, 1, 1), 0)
    l1 = jax.lax.broadcasted_iota(jnp.int32, (1, 128), 1)
    r2 = jax.lax.broadcasted_iota(jnp.int32, (1, RS, 128), 1)
    l2 = jax.lax.broadcasted_iota(jnp.int32, (1, RS, 128), 2)
    accs = [[jnp.zeros((1, K), jnp.float32), jnp.zeros((1, K), jnp.int32),
             jnp.zeros((1, K), jnp.int32)] for _ in range(BP)]
    for i in range(K):
        for b in range(BP):
            g = jnp.max(cs[b])
            fi = jnp.min(jnp.where(cs[b] == g, fs[b], BIG_I))
            bi = fi >> 15
            v = fi & (V - 1)
            s = v >> SEG_SHIFT
            r = (v >> 7) & (RS - 1)
            lw = v & 127
            bq = b * BEAM + bi
            accs[b][0] = jnp.where(oiota == i, g, accs[b][0])
            accs[b][1] = jnp.where(oiota == i, bi, accs[b][1])
            accs[b][2] = jnp.where(oiota == i, v, accs[b][2])
            row = lp_ref[bq, s, pl.ds(r, 1), :]
            lp_ref[bq, s, pl.ds(r, 1), :] = jnp.where(
                l1 == lw, jnp.float32(NEG_SENT), row)
            xq = lp_ref[bq, pl.ds(s, 1)]
            crow = jnp.max(xq, axis=1)
            frow = jnp.min(jnp.where(xq == crow[:, None, :],
                                     bi * V + s * (RS * 128) + r2 * 128 + l2,
                                     BIG_I), axis=1)
            shift_b = jnp.max(jnp.where(rowsNB == bq, shift,
                                        jnp.float32(NEG_SENT)))
            sel = (i16 == bi) & (is4 == s)
            cs[b] = jnp.where(sel, (crow + shift_b)[:, None, :], cs[b])
            fs[b] = jnp.where(sel, frow[:, None, :], fs[b])

    for b in range(BP):
        vals_ref[b] = accs[b][0]
        beam_ref[b] = accs[b][1]
        tok_ref[b] = accs[b][2]


def kernel(log_probs, alive_scores, k):
    del k
    lp4 = log_probs.reshape(B * BEAM, SEG, RS, 128)
    alive3 = jnp.broadcast_to(alive_scores[..., None], (B, BEAM, 128))
    out_shapes = [
        jax.ShapeDtypeStruct((B, 1, K), jnp.float32),
        jax.ShapeDtypeStruct((B, 1, K), jnp.int32),
        jax.ShapeDtypeStruct((B, 1, K), jnp.int32),
    ]
    out_spec = pl.BlockSpec((BP, 1, K), lambda b: (b, 0, 0))
    topv, beam_idx, tok_idx = pl.pallas_call(
        _beam_step_kernel,
        grid=(B // BP,),
        in_specs=[
            pl.BlockSpec((NB, SEG, RS, 128), lambda b: (b, 0, 0, 0)),
            pl.BlockSpec((BP, BEAM, 128), lambda b: (b, 0, 0)),
        ],
        out_specs=[out_spec, out_spec, out_spec],
        out_shape=out_shapes,
    )(lp4, alive3)
    return (topv.reshape(B, K), beam_idx.reshape(B, K), tok_idx.reshape(B, K))
